# Initial kernel scaffold; baseline (speedup 1.0000x reference)
import jax
import jax.numpy as jnp
from jax.experimental import pallas as pl


def kernel(lut, x):
    raise NotImplementedError("write your pallas kernel here")


# speedup vs baseline: 1486.2829x; 1486.2829x over previous
import functools

import jax
import jax.numpy as jnp
import numpy as np
from jax import lax
from jax.experimental import pallas as pl
from jax.experimental.pallas import tpu as pltpu
from jax.experimental.pallas import tpu_sc as plsc

DIM = 33
SHIFT = DIM ** 3
TBL = 3 * SHIFT
TBL_PAD = 107824
NPX = 512 * 512
NBATCH = 4
CHUNK = 2048
L = 16
NWORKERS = 32

_INV_BS = np.float32(1.0 / (1.000001 / (DIM - 1)))
_OFFS = (0, 1, DIM, DIM + 1, DIM * DIM, DIM * DIM + 1,
         DIM * DIM + DIM, DIM * DIM + DIM + 1)


def _body(lut_hbm, x_hbm, out_hbm, tbl, rin, gin, bin_, rout, gout, bout):
    nc = lax.axis_size("c")
    wid = lax.axis_index("s") * nc + lax.axis_index("c")
    pltpu.sync_copy(lut_hbm, tbl)

    px_per_w = NBATCH * NPX // NWORKERS
    w_per_batch = NPX // px_per_w
    batch = wid // w_per_batch
    plane_off = (wid % w_per_batch) * px_per_w

    def chunk_body(t, carry):
        off = plane_off + t * CHUNK
        rbase = (batch * 3 + 0) * NPX + off
        gbase = (batch * 3 + 1) * NPX + off
        bbase = (batch * 3 + 2) * NPX + off
        pltpu.sync_copy(x_hbm.at[pl.ds(rbase, CHUNK)], rin)
        pltpu.sync_copy(x_hbm.at[pl.ds(gbase, CHUNK)], gin)
        pltpu.sync_copy(x_hbm.at[pl.ds(bbase, CHUNK)], bin_)

        def vec_body(j, carry2):
            s = j * L
            rv = rin[pl.ds(s, L)]
            gv = gin[pl.ds(s, L)]
            bv = bin_[pl.ds(s, L)]
            ridx = rv * _INV_BS
            gidx = gv * _INV_BS
            bidx = bv * _INV_BS
            rid = jnp.minimum(ridx.astype(jnp.int32), DIM - 2)
            gid = jnp.minimum(gidx.astype(jnp.int32), DIM - 2)
            bid = jnp.minimum(bidx.astype(jnp.int32), DIM - 2)
            rd = ridx - rid.astype(jnp.float32)
            gd = gidx - gid.astype(jnp.float32)
            bd = bidx - bid.astype(jnp.float32)
            id000 = rid + gid * DIM + bid * (DIM * DIM)
            r1 = jnp.float32(1) - rd
            g1 = jnp.float32(1) - gd
            b1 = jnp.float32(1) - bd
            pg0 = g1 * b1
            pg1 = gd * b1
            pg2 = g1 * bd
            pg3 = gd * bd
            ws = (r1 * pg0, rd * pg0, r1 * pg1, rd * pg1,
                  r1 * pg2, rd * pg2, r1 * pg3, rd * pg3)
            for c, obuf in ((0, rout), (1, gout), (2, bout)):
                base = id000 + c * SHIFT
                acc = ws[0] * plsc.load_gather(tbl, [base])
                for k in range(1, 8):
                    acc = acc + ws[k] * plsc.load_gather(tbl, [base + _OFFS[k]])
                obuf[pl.ds(s, L)] = acc
            return carry2

        lax.fori_loop(0, CHUNK // L, vec_body, 0, unroll=False)
        pltpu.sync_copy(rout, out_hbm.at[pl.ds(rbase, CHUNK)])
        pltpu.sync_copy(gout, out_hbm.at[pl.ds(gbase, CHUNK)])
        pltpu.sync_copy(bout, out_hbm.at[pl.ds(bbase, CHUNK)])
        return carry

    lax.fori_loop(0, px_per_w // CHUNK, chunk_body, 0, unroll=False)


@jax.jit
def _run(lut_pad, xf):
    mesh = plsc.VectorSubcoreMesh(core_axis_name="c", subcore_axis_name="s")
    f = pl.kernel(
        _body,
        out_type=jax.ShapeDtypeStruct((NBATCH * 3 * NPX,), jnp.float32),
        mesh=mesh,
        compiler_params=pltpu.CompilerParams(needs_layout_passes=False),
        scratch_types=[
            pltpu.VMEM((TBL_PAD,), jnp.float32),
            pltpu.VMEM((CHUNK,), jnp.float32),
            pltpu.VMEM((CHUNK,), jnp.float32),
            pltpu.VMEM((CHUNK,), jnp.float32),
            pltpu.VMEM((CHUNK,), jnp.float32),
            pltpu.VMEM((CHUNK,), jnp.float32),
            pltpu.VMEM((CHUNK,), jnp.float32),
        ],
    )
    return f(lut_pad, xf)


def kernel(lut, x):
    lut_pad = jnp.pad(lut.reshape(TBL), (0, TBL_PAD - TBL))
    out_flat = _run(lut_pad, x.reshape(-1))
    return out_flat.reshape(x.shape)


# speedup vs baseline: 1671.8868x; 1.1249x over previous
import jax
import jax.numpy as jnp
import numpy as np
from jax import lax
from jax.experimental import pallas as pl
from jax.experimental.pallas import tpu as pltpu
from jax.experimental.pallas import tpu_sc as plsc

DIM = 33
SHIFT = DIM ** 3
TBL = 3 * SHIFT
NPX = 512 * 512
NBATCH = 4
CHUNK = 1024
L = 16
NWORKERS = 32
PX_PER_W = NBATCH * NPX // NWORKERS
NCHUNKS = PX_PER_W // CHUNK

_INV_BS = np.float32(1.0 / (1.000001 / (DIM - 1)))
_OFFS = (0, 1, DIM, DIM + 1, DIM * DIM, DIM * DIM + 1,
         DIM * DIM + DIM, DIM * DIM + DIM + 1)


def _body(lut_hbm, x_hbm, out_hbm, tbl,
          rin0, gin0, bin0, rin1, gin1, bin1,
          rout0, gout0, bout0, rout1, gout1, bout1,
          tsem, lsem0, lsem1, ssem0, ssem1):
    ins = ((rin0, gin0, bin0), (rin1, gin1, bin1))
    outs = ((rout0, gout0, bout0), (rout1, gout1, bout1))
    lsems = (lsem0, lsem1)
    ssems = (ssem0, ssem1)
    nc = lax.axis_size("c")
    wid = lax.axis_index("s") * nc + lax.axis_index("c")

    w_per_batch = NPX // PX_PER_W
    batch = wid // w_per_batch
    plane_off = (wid % w_per_batch) * PX_PER_W
    rbase = (batch * 3 + 0) * NPX + plane_off
    gbase = (batch * 3 + 1) * NPX + plane_off
    bbase = (batch * 3 + 2) * NPX + plane_off

    tbl_cp = pltpu.async_copy(lut_hbm, tbl, tsem)

    def issue_loads(t, slot):
        o = t * CHUNK
        return [
            pltpu.async_copy(x_hbm.at[pl.ds(rbase + o, CHUNK)],
                             ins[slot][0], lsems[slot]),
            pltpu.async_copy(x_hbm.at[pl.ds(gbase + o, CHUNK)],
                             ins[slot][1], lsems[slot]),
            pltpu.async_copy(x_hbm.at[pl.ds(bbase + o, CHUNK)],
                             ins[slot][2], lsems[slot]),
        ]

    def issue_stores(t, slot):
        o = t * CHUNK
        return [
            pltpu.async_copy(outs[slot][0],
                             out_hbm.at[pl.ds(rbase + o, CHUNK)], ssems[slot]),
            pltpu.async_copy(outs[slot][1],
                             out_hbm.at[pl.ds(gbase + o, CHUNK)], ssems[slot]),
            pltpu.async_copy(outs[slot][2],
                             out_hbm.at[pl.ds(bbase + o, CHUNK)], ssems[slot]),
        ]

    def compute_chunk(slot):
        def vec_body(j, carry):
            s = j * L
            rv = ins[slot][0][pl.ds(s, L)]
            gv = ins[slot][1][pl.ds(s, L)]
            bv = ins[slot][2][pl.ds(s, L)]
            ridx = rv * _INV_BS
            gidx = gv * _INV_BS
            bidx = bv * _INV_BS
            rid = jnp.minimum(ridx.astype(jnp.int32), DIM - 2)
            gid = jnp.minimum(gidx.astype(jnp.int32), DIM - 2)
            bid = jnp.minimum(bidx.astype(jnp.int32), DIM - 2)
            rd = ridx - rid.astype(jnp.float32)
            gd = gidx - gid.astype(jnp.float32)
            bd = bidx - bid.astype(jnp.float32)
            id000 = rid + gid * DIM + bid * (DIM * DIM)
            r1 = jnp.float32(1) - rd
            g1 = jnp.float32(1) - gd
            b1 = jnp.float32(1) - bd
            pg0 = g1 * b1
            pg1 = gd * b1
            pg2 = g1 * bd
            pg3 = gd * bd
            ws = (r1 * pg0, rd * pg0, r1 * pg1, rd * pg1,
                  r1 * pg2, rd * pg2, r1 * pg3, rd * pg3)
            for c in range(3):
                base = id000 + c * SHIFT
                acc = ws[0] * plsc.load_gather(tbl, [base])
                for k in range(1, 8):
                    acc = acc + ws[k] * plsc.load_gather(tbl, [base + _OFFS[k]])
                outs[slot][c][pl.ds(s, L)] = acc
            return carry

        lax.fori_loop(0, CHUNK // L, vec_body, 0, unroll=False)

    pending_loads = issue_loads(0, 0)
    tbl_cp.wait()

    pending_stores = [None, None]
    for t in range(NCHUNKS):
        slot = t % 2
        for cp in pending_loads:
            cp.wait()
        if t + 1 < NCHUNKS:
            pending_loads = issue_loads(t + 1, 1 - slot)
        if pending_stores[slot] is not None:
            for cp in pending_stores[slot]:
                cp.wait()
        compute_chunk(slot)
        pending_stores[slot] = issue_stores(t, slot)
    for ps in pending_stores:
        for cp in ps:
            cp.wait()


@jax.jit
def _run(lut_flat, xf):
    mesh = plsc.VectorSubcoreMesh(core_axis_name="c", subcore_axis_name="s")
    f = pl.kernel(
        _body,
        out_type=jax.ShapeDtypeStruct((NBATCH * 3 * NPX,), jnp.float32),
        mesh=mesh,
        compiler_params=pltpu.CompilerParams(needs_layout_passes=False),
        scratch_types=[
            pltpu.VMEM((TBL,), jnp.float32),
        ] + [pltpu.VMEM((CHUNK,), jnp.float32)] * 12 + [
            pltpu.SemaphoreType.DMA,
            pltpu.SemaphoreType.DMA,
            pltpu.SemaphoreType.DMA,
            pltpu.SemaphoreType.DMA,
            pltpu.SemaphoreType.DMA,
        ],
    )
    return f(lut_flat, xf)


def kernel(lut, x):
    out_flat = _run(lut.reshape(TBL), x.reshape(-1))
    return out_flat.reshape(x.shape)
